# baseline (speedup 1.0000x reference)
import functools
import math

import jax
import jax.numpy as jnp
from jax import lax
from jax.experimental import pallas as pl
from jax.experimental.pallas import tpu as pltpu
from jax.experimental.pallas import tpu_sc as plsc

D_EMBED = 64
SCALE = math.sqrt(D_EMBED)
LANES = 16
CHUNK = 256


def _gather(idx_flat, table_scaled):
    n = idx_flat.shape[0]
    info = plsc.get_sparse_core_info()
    num_workers = info.num_cores * info.num_subcores
    per_w = n // num_workers
    n_chunks = per_w // CHUNK

    mesh = plsc.VectorSubcoreMesh(core_axis_name="c", subcore_axis_name="s")

    @functools.partial(
        pl.kernel,
        mesh=mesh,
        out_type=jax.ShapeDtypeStruct((n, D_EMBED), jnp.float32),
        scratch_types=[
            pltpu.VMEM((2, CHUNK), jnp.int32),
            pltpu.VMEM((2, CHUNK, D_EMBED), jnp.float32),
            pltpu.SemaphoreType.DMA((2,)),
            pltpu.SemaphoreType.DMA((2,)),
        ],
    )
    def emb(idx_hbm, table_hbm, out_hbm, idx_v, rows_v, gsem, wsem):
        wid = lax.axis_index("s") * info.num_cores + lax.axis_index("c")
        base = wid * per_w

        def fire(g):
            b = lax.rem(g, 2)
            off = base + g * CHUNK
            pltpu.sync_copy(idx_hbm.at[pl.ds(off, CHUNK)], idx_v.at[b])

            def fire_body(k, c):
                vec = idx_v[b, pl.ds(k * LANES, LANES)]
                for j in range(LANES):
                    pltpu.async_copy(
                        table_hbm.at[vec[j]],
                        rows_v.at[b, k * LANES + j],
                        gsem.at[b],
                    )
                return c

            lax.fori_loop(0, CHUNK // LANES, fire_body, 0)

        def drain(g):
            b = lax.rem(g, 2)
            pltpu.make_async_copy(
                table_hbm.at[pl.ds(0, CHUNK)], rows_v.at[b], gsem.at[b]
            ).wait()

        def write_start(g):
            b = lax.rem(g, 2)
            off = base + g * CHUNK
            pltpu.async_copy(
                rows_v.at[b], out_hbm.at[pl.ds(off, CHUNK)], wsem.at[b]
            )

        def write_wait(g):
            b = lax.rem(g, 2)
            pltpu.make_async_copy(
                rows_v.at[b], out_hbm.at[pl.ds(0, CHUNK)], wsem.at[b]
            ).wait()

        def scale(g):
            b = lax.rem(g, 2)

            def scale_body(i, c):
                for j in range(D_EMBED // LANES):
                    rows_v[b, i, pl.ds(j * LANES, LANES)] = (
                        rows_v[b, i, pl.ds(j * LANES, LANES)] * SCALE
                    )
                return c

            lax.fori_loop(0, CHUNK, scale_body, 0)

        fire(0)

        def body(g, carry):
            @pl.when(g >= 1)
            def _():
                write_wait(g - 1)

            fire(g + 1)
            drain(g)
            scale(g)
            write_start(g)
            return carry

        lax.fori_loop(0, n_chunks - 1, body, 0)

        g_last = n_chunks - 1
        drain(g_last)
        scale(g_last)
        write_start(g_last)
        write_wait(g_last - 1)
        write_wait(g_last)

    return emb(idx_flat, table_scaled)


def kernel(x, table):
    batch, seq = x.shape
    idx_flat = x.reshape(-1).astype(jnp.int32)
    out = _gather(idx_flat, table)
    return out.reshape(batch, seq, D_EMBED)


# speedup vs baseline: 1.5653x; 1.5653x over previous
import functools
import math

import jax
import jax.numpy as jnp
from jax import lax
from jax.experimental import pallas as pl
from jax.experimental.pallas import tpu as pltpu
from jax.experimental.pallas import tpu_sc as plsc

D_EMBED = 64
SCALE = math.sqrt(D_EMBED)
LANES = 16
CHUNK = 256


def _gather_scale(idx_flat, table):
    n = idx_flat.shape[0]
    info = plsc.get_sparse_core_info()
    num_workers = info.num_cores * info.num_subcores
    per_w = n // num_workers
    n_chunks = per_w // CHUNK
    n_pairs = n_chunks // 2

    mesh = plsc.VectorSubcoreMesh(core_axis_name="c", subcore_axis_name="s")

    @functools.partial(
        pl.kernel,
        mesh=mesh,
        out_type=jax.ShapeDtypeStruct((n, D_EMBED), jnp.float32),
        scratch_types=[
            pltpu.VMEM((2, CHUNK), jnp.int32),
            pltpu.VMEM((2, CHUNK, D_EMBED), jnp.float32),
            pltpu.SemaphoreType.DMA,
            pltpu.SemaphoreType.DMA,
            pltpu.SemaphoreType.DMA,
            pltpu.SemaphoreType.DMA,
            pltpu.SemaphoreType.DMA,
            pltpu.SemaphoreType.DMA,
        ],
    )
    def emb(idx_hbm, table_hbm, out_hbm, idx_v, rows_v, g0, g1, w0, w1, i0, i1):
        gsems = (g0, g1)
        wsems = (w0, w1)
        isems = (i0, i1)
        wid = lax.axis_index("s") * info.num_cores + lax.axis_index("c")
        base = wid * per_w

        def idx_start(g, b):
            off = base + g * CHUNK
            pltpu.async_copy(
                idx_hbm.at[pl.ds(off, CHUNK)], idx_v.at[b], isems[b]
            )

        def idx_wait(b):
            pltpu.make_async_copy(
                idx_hbm.at[pl.ds(0, CHUNK)], idx_v.at[b], isems[b]
            ).wait()

        def fire(b):
            def fire_body(k, c):
                vec = idx_v[b, pl.ds(k * LANES, LANES)]
                for j in range(LANES):
                    pltpu.async_copy(
                        table_hbm.at[vec[j]],
                        rows_v.at[b, k * LANES + j],
                        gsems[b],
                    )
                return c

            lax.fori_loop(0, CHUNK // LANES, fire_body, 0)

        def drain(b):
            pltpu.make_async_copy(
                table_hbm.at[pl.ds(0, CHUNK)], rows_v.at[b], gsems[b]
            ).wait()

        def scale(b):
            def scale_body(i, c):
                for j in range(D_EMBED // LANES):
                    rows_v[b, i, pl.ds(j * LANES, LANES)] = (
                        rows_v[b, i, pl.ds(j * LANES, LANES)] * SCALE
                    )
                return c

            lax.fori_loop(0, CHUNK, scale_body, 0)

        def write_start(g, b):
            off = base + g * CHUNK
            pltpu.async_copy(
                rows_v.at[b], out_hbm.at[pl.ds(off, CHUNK)], wsems[b]
            )

        def write_wait(b):
            pltpu.make_async_copy(
                rows_v.at[b], out_hbm.at[pl.ds(0, CHUNK)], wsems[b]
            ).wait()

        idx_start(0, 0)
        idx_start(1, 1)
        idx_wait(0)
        fire(0)

        def body(m, carry):
            e = 2 * m
            o = e + 1
            drain(0)

            @pl.when(m >= 1)
            def _():
                write_wait(1)

            idx_wait(1)
            fire(1)

            @pl.when(m + 1 < n_pairs)
            def _():
                idx_start(e + 2, 0)

            scale(0)
            write_start(e, 0)
            drain(1)
            write_wait(0)

            @pl.when(m + 1 < n_pairs)
            def _():
                idx_wait(0)
                fire(0)
                idx_start(o + 2, 1)

            scale(1)
            write_start(o, 1)
            return carry

        lax.fori_loop(0, n_pairs, body, 0)
        write_wait(1)

    return emb(idx_flat, table)


def kernel(x, table):
    batch, seq = x.shape
    idx_flat = x.reshape(-1).astype(jnp.int32)
    out = _gather_scale(idx_flat, table)
    return out.reshape(batch, seq, D_EMBED)


# speedup vs baseline: 1.6475x; 1.0525x over previous
import functools
import math

import jax
import jax.numpy as jnp
from jax import lax
from jax.experimental import pallas as pl
from jax.experimental.pallas import tpu as pltpu
from jax.experimental.pallas import tpu_sc as plsc

D_EMBED = 64
SCALE = math.sqrt(D_EMBED)
LANES = 16
CHUNK = 256


def _gather_scale(idx_flat, table):
    n = idx_flat.shape[0]
    info = plsc.get_sparse_core_info()
    num_workers = info.num_cores * info.num_subcores
    per_w = n // num_workers
    n_chunks = per_w // CHUNK
    n_pairs = n_chunks // 2

    mesh = plsc.VectorSubcoreMesh(core_axis_name="c", subcore_axis_name="s")

    @functools.partial(
        pl.kernel,
        mesh=mesh,
        out_type=jax.ShapeDtypeStruct((n, D_EMBED), jnp.float32),
        scratch_types=[
            pltpu.VMEM((2, CHUNK), jnp.int32),
            pltpu.VMEM((2, CHUNK, D_EMBED), jnp.float32),
            pltpu.SemaphoreType.DMA,
            pltpu.SemaphoreType.DMA,
            pltpu.SemaphoreType.DMA,
            pltpu.SemaphoreType.DMA,
            pltpu.SemaphoreType.DMA,
            pltpu.SemaphoreType.DMA,
        ],
    )
    def emb(idx_hbm, table_hbm, out_hbm, idx_v, rows_v, g0, g1, w0, w1, i0, i1):
        gsems = (g0, g1)
        wsems = (w0, w1)
        isems = (i0, i1)
        wid = lax.axis_index("s") * info.num_cores + lax.axis_index("c")
        base = wid * per_w

        def idx_start(g, b):
            off = base + g * CHUNK
            pltpu.async_copy(
                idx_hbm.at[pl.ds(off, CHUNK)], idx_v.at[b], isems[b]
            )

        def idx_wait(b):
            pltpu.make_async_copy(
                idx_hbm.at[pl.ds(0, CHUNK)], idx_v.at[b], isems[b]
            ).wait()

        def fire(b):
            def fire_body(k, c):
                vec = idx_v[b, pl.ds(k * LANES, LANES)]
                for j in range(LANES):
                    pltpu.async_copy(
                        table_hbm.at[vec[j]],
                        rows_v.at[b, k * LANES + j],
                        gsems[b],
                    )
                return c

            lax.fori_loop(0, CHUNK // LANES, fire_body, 0, unroll=2)

        def drain(b):
            pltpu.make_async_copy(
                table_hbm.at[pl.ds(0, CHUNK)], rows_v.at[b], gsems[b]
            ).wait()

        def scale(b):
            def scale_body(i, c):
                for j in range(D_EMBED // LANES):
                    rows_v[b, i, pl.ds(j * LANES, LANES)] = (
                        rows_v[b, i, pl.ds(j * LANES, LANES)] * SCALE
                    )
                return c

            lax.fori_loop(0, CHUNK, scale_body, 0, unroll=4)

        def write_start(g, b):
            off = base + g * CHUNK
            pltpu.async_copy(
                rows_v.at[b], out_hbm.at[pl.ds(off, CHUNK)], wsems[b]
            )

        def write_wait(b):
            pltpu.make_async_copy(
                rows_v.at[b], out_hbm.at[pl.ds(0, CHUNK)], wsems[b]
            ).wait()

        idx_start(0, 0)
        idx_start(1, 1)
        idx_wait(0)
        fire(0)

        def body(m, carry):
            e = 2 * m
            o = e + 1
            drain(0)

            @pl.when(m >= 1)
            def _():
                write_wait(1)

            idx_wait(1)
            fire(1)

            @pl.when(m + 1 < n_pairs)
            def _():
                idx_start(e + 2, 0)

            scale(0)
            write_start(e, 0)
            drain(1)
            write_wait(0)

            @pl.when(m + 1 < n_pairs)
            def _():
                idx_wait(0)
                fire(0)
                idx_start(o + 2, 1)

            scale(1)
            write_start(o, 1)
            return carry

        lax.fori_loop(0, n_pairs, body, 0)
        write_wait(1)

    return emb(idx_flat, table)


def kernel(x, table):
    batch, seq = x.shape
    idx_flat = x.reshape(-1).astype(jnp.int32)
    out = _gather_scale(idx_flat, table)
    return out.reshape(batch, seq, D_EMBED)
